# baseline (speedup 1.0000x reference)
import jax
import jax.numpy as jnp
from jax.experimental import pallas as pl
from jax.experimental.pallas import tpu as pltpu

TOPK = 8
NEG = -1e9
NEGINF = -3.0e38


def _body(x_ref, labc_ref, labr_ref, w1t_ref, w2t_ref, g_ref, b_ref,
          wpt_ref, out_ref, kprev, mprev, pprev):
    i = pl.program_id(0)
    X = x_ref[0]
    B = X.shape[0]
    labc = labc_ref[...]
    labr = labr_ref[...]

    nrm = jnp.sqrt(jnp.sum(X * X, axis=1, keepdims=True))
    Xn = X / jnp.maximum(nrm, 1e-8)
    S = jax.lax.dot_general(Xn, Xn, (((1,), (1,)), ((), ())),
                            preferred_element_type=jnp.float32)
    S = jnp.clip(S, -1.0 + 1e-8, 1.0 - 1e-8)

    rows = jax.lax.broadcasted_iota(jnp.int32, (B, B), 0)
    cols = jax.lax.broadcasted_iota(jnp.int32, (B, B), 1)
    eye = rows == cols
    same = labc == labr
    allowed = same & (~eye)
    masked = jnp.where(allowed, S, NEG)

    w = masked
    for t in range(TOPK):
        tc = jnp.max(w, axis=1, keepdims=True)
        if t < TOPK - 1:
            w = jnp.where(w >= tc, NEGINF, w)
    eyef = jnp.where(eye, 1.0, 0.0)
    tr = jax.lax.dot_general(tc, eyef, (((0,), (0,)), ((), ())),
                             precision=jax.lax.Precision.HIGHEST,
                             preferred_element_type=jnp.float32)

    msym = ((S >= tc) | (S >= tr)) & allowed
    Mf = jnp.where(msym, 1.0, 0.0).astype(jnp.float32)

    A = Mf * jnp.maximum(S, 0.0) + 1e-6 * eyef
    dinv_c = jax.lax.rsqrt(jnp.maximum(jnp.sum(A, axis=1, keepdims=True), 1e-8))
    dinv_r = jax.lax.dot_general(dinv_c, eyef, (((0,), (0,)), ((), ())),
                                 precision=jax.lax.Precision.HIGHEST,
                                 preferred_element_type=jnp.float32)
    A_norm = A * dinv_c * dinv_r

    W1t = w1t_ref[0]
    W2t = w2t_ref[0]
    gam = g_ref[0]
    bet = b_ref[0]
    bf = jnp.bfloat16
    Ab = A_norm.astype(bf)
    Z = jnp.dot(Ab, jnp.dot(X.astype(bf), W1t,
                            preferred_element_type=jnp.float32).astype(bf),
                preferred_element_type=jnp.float32)
    mu = jnp.mean(Z, axis=0, keepdims=True)
    var = jnp.mean((Z - mu) ** 2, axis=0, keepdims=True)
    Z = (Z - mu) * jax.lax.rsqrt(var + 1e-5) * gam + bet
    Z = jnp.maximum(Z, 0.0)
    Z = jnp.dot(Ab, jnp.dot(Z.astype(bf), W2t,
                            preferred_element_type=jnp.float32).astype(bf),
                preferred_element_type=jnp.float32) + X

    Pj = jnp.dot(Z.astype(bf), wpt_ref[...],
                 preferred_element_type=jnp.float32)
    pn = jnp.sqrt(jnp.sum(Pj * Pj, axis=1, keepdims=True))
    Pn = Pj / jnp.maximum(pn, 1e-8)

    @pl.when(i == 0)
    def _init():
        out_ref[...] = jnp.zeros_like(out_ref)

    @pl.when(i > 0)
    def _acc():
        Me = jnp.maximum(mprev[...], Mf)
        num = jnp.sum((kprev[...] - A_norm) ** 2 * Me)
        den = jnp.maximum(jnp.sum(Me), 1e-8)
        lk = num / den
        lz = jnp.sum((pprev[...] - Pn) ** 2) / (Pn.shape[0] * Pn.shape[1])
        lane = jax.lax.broadcasted_iota(jnp.int32, out_ref.shape, 1)
        out_ref[...] += (jnp.where(lane == 0, lk, 0.0)
                         + jnp.where(lane == 1, lz, 0.0)
                         + jnp.where(lane == 2, 64.0 * lk + 16.0 * lz, 0.0))

    kprev[...] = A_norm
    mprev[...] = Mf
    pprev[...] = Pn


def kernel(feats_final, labels, W1s, W2s, gammas, betas, Wproj):
    L, B, D = feats_final.shape
    P = Wproj.shape[0]

    labc = labels.astype(jnp.int32).reshape(B, 1)
    labr = labels.astype(jnp.int32).reshape(1, B)
    W1t = W1s.transpose(0, 2, 1).astype(jnp.bfloat16)
    W2t = W2s.transpose(0, 2, 1).astype(jnp.bfloat16)
    Wpt = Wproj.T.astype(jnp.bfloat16)
    g3 = gammas.reshape(L, 1, D)
    b3 = betas.reshape(L, 1, D)

    acc = pl.pallas_call(
        _body,
        grid=(L,),
        in_specs=[
            pl.BlockSpec((1, B, D), lambda i: (i, 0, 0)),
            pl.BlockSpec((B, 1), lambda i: (0, 0)),
            pl.BlockSpec((1, B), lambda i: (0, 0)),
            pl.BlockSpec((1, D, D), lambda i: (i, 0, 0)),
            pl.BlockSpec((1, D, D), lambda i: (i, 0, 0)),
            pl.BlockSpec((1, 1, D), lambda i: (i, 0, 0)),
            pl.BlockSpec((1, 1, D), lambda i: (i, 0, 0)),
            pl.BlockSpec((D, P), lambda i: (0, 0)),
        ],
        out_specs=pl.BlockSpec((1, 128), lambda i: (0, 0)),
        out_shape=jax.ShapeDtypeStruct((1, 128), jnp.float32),
        scratch_shapes=[
            pltpu.VMEM((B, B), jnp.float32),
            pltpu.VMEM((B, B), jnp.float32),
            pltpu.VMEM((B, P), jnp.float32),
        ],
    )(feats_final, labc, labr, W1t, W2t, g3, b3, Wpt)

    return (acc[0, 0], acc[0, 1], acc[0, 2])


# speedup vs baseline: 1.0676x; 1.0676x over previous
import jax
import jax.numpy as jnp
from jax.experimental import pallas as pl
from jax.experimental.pallas import tpu as pltpu

TOPK = 8
NEG = -1e9
NEGINF = -3.0e38


def _body(x_ref, labc_ref, labr_ref, w1t_ref, w2t_ref, g_ref, b_ref,
          wpt_ref, out_ref, kprev, mprev, pprev):
    i = pl.program_id(0)
    X = x_ref[0]
    B = X.shape[0]
    labc = labc_ref[...]
    labr = labr_ref[...]

    nrm = jnp.sqrt(jnp.sum(X * X, axis=1, keepdims=True))
    Xn = X / jnp.maximum(nrm, 1e-8)
    S = jax.lax.dot_general(Xn, Xn, (((1,), (1,)), ((), ())),
                            preferred_element_type=jnp.float32)
    S = jnp.clip(S, -1.0 + 1e-8, 1.0 - 1e-8)

    rows = jax.lax.broadcasted_iota(jnp.int32, (B, B), 0)
    cols = jax.lax.broadcasted_iota(jnp.int32, (B, B), 1)
    eye = rows == cols
    same = labc == labr
    allowed = same & (~eye)
    masked = jnp.where(allowed, S, NEG)

    w = masked
    for t in range(TOPK):
        tc = jnp.max(w, axis=1, keepdims=True)
        if t < TOPK - 1:
            w = jnp.where(w >= tc, NEGINF, w)
    eyef = jnp.where(eye, 1.0, 0.0)
    tr = jnp.transpose(tc)

    msym = ((S >= tc) | (S >= tr)) & allowed
    Mf = jnp.where(msym, 1.0, 0.0).astype(jnp.float32)

    A = Mf * jnp.maximum(S, 0.0) + 1e-6 * eyef
    dinv_c = jax.lax.rsqrt(jnp.maximum(jnp.sum(A, axis=1, keepdims=True), 1e-8))
    dinv_r = jax.lax.rsqrt(jnp.maximum(jnp.sum(A, axis=0, keepdims=True), 1e-8))
    A_norm = A * dinv_c * dinv_r

    W1t = w1t_ref[0]
    W2t = w2t_ref[0]
    gam = g_ref[0]
    bet = b_ref[0]
    bf = jnp.bfloat16
    Ab = A_norm.astype(bf)
    Z = jnp.dot(Ab, jnp.dot(X.astype(bf), W1t,
                            preferred_element_type=jnp.float32).astype(bf),
                preferred_element_type=jnp.float32)
    mu = jnp.mean(Z, axis=0, keepdims=True)
    var = jnp.mean((Z - mu) ** 2, axis=0, keepdims=True)
    Z = (Z - mu) * jax.lax.rsqrt(var + 1e-5) * gam + bet
    Z = jnp.maximum(Z, 0.0)
    Z = jnp.dot(Ab, jnp.dot(Z.astype(bf), W2t,
                            preferred_element_type=jnp.float32).astype(bf),
                preferred_element_type=jnp.float32) + X

    Pj = jnp.dot(Z.astype(bf), wpt_ref[...],
                 preferred_element_type=jnp.float32)
    pn = jnp.sqrt(jnp.sum(Pj * Pj, axis=1, keepdims=True))
    Pn = Pj / jnp.maximum(pn, 1e-8)

    @pl.when(i == 0)
    def _init():
        out_ref[...] = jnp.zeros_like(out_ref)

    @pl.when(i > 0)
    def _acc():
        Me = jnp.maximum(mprev[...], Mf)
        num = jnp.sum((kprev[...] - A_norm) ** 2 * Me)
        den = jnp.maximum(jnp.sum(Me), 1e-8)
        lk = num / den
        lz = jnp.sum((pprev[...] - Pn) ** 2) / (Pn.shape[0] * Pn.shape[1])
        lane = jax.lax.broadcasted_iota(jnp.int32, out_ref.shape, 1)
        out_ref[...] += (jnp.where(lane == 0, lk, 0.0)
                         + jnp.where(lane == 1, lz, 0.0)
                         + jnp.where(lane == 2, 64.0 * lk + 16.0 * lz, 0.0))

    kprev[...] = A_norm
    mprev[...] = Mf
    pprev[...] = Pn


def kernel(feats_final, labels, W1s, W2s, gammas, betas, Wproj):
    L, B, D = feats_final.shape
    P = Wproj.shape[0]

    labc = labels.astype(jnp.int32).reshape(B, 1)
    labr = labels.astype(jnp.int32).reshape(1, B)
    W1t = W1s.transpose(0, 2, 1).astype(jnp.bfloat16)
    W2t = W2s.transpose(0, 2, 1).astype(jnp.bfloat16)
    Wpt = Wproj.T.astype(jnp.bfloat16)
    g3 = gammas.reshape(L, 1, D)
    b3 = betas.reshape(L, 1, D)

    acc = pl.pallas_call(
        _body,
        grid=(L,),
        in_specs=[
            pl.BlockSpec((1, B, D), lambda i: (i, 0, 0)),
            pl.BlockSpec((B, 1), lambda i: (0, 0)),
            pl.BlockSpec((1, B), lambda i: (0, 0)),
            pl.BlockSpec((1, D, D), lambda i: (i, 0, 0)),
            pl.BlockSpec((1, D, D), lambda i: (i, 0, 0)),
            pl.BlockSpec((1, 1, D), lambda i: (i, 0, 0)),
            pl.BlockSpec((1, 1, D), lambda i: (i, 0, 0)),
            pl.BlockSpec((D, P), lambda i: (0, 0)),
        ],
        out_specs=pl.BlockSpec((1, 128), lambda i: (0, 0)),
        out_shape=jax.ShapeDtypeStruct((1, 128), jnp.float32),
        scratch_shapes=[
            pltpu.VMEM((B, B), jnp.float32),
            pltpu.VMEM((B, B), jnp.float32),
            pltpu.VMEM((B, P), jnp.float32),
        ],
    )(feats_final, labc, labr, W1t, W2t, g3, b3, Wpt)

    return (acc[0, 0], acc[0, 1], acc[0, 2])


# speedup vs baseline: 1.0800x; 1.0116x over previous
import jax
import jax.numpy as jnp
from jax.experimental import pallas as pl
from jax.experimental.pallas import tpu as pltpu

TOPK = 8
NEG = -1e9
NEGINF = -3.0e38


def _body(x_ref, labc_ref, labr_ref, w1t_ref, w2t_ref, g_ref, b_ref,
          wpt_ref, out_ref, kprev, mprev, pprev):
    i = pl.program_id(0)
    X = x_ref[0]
    B = X.shape[0]
    labc = labc_ref[...]
    labr = labr_ref[...]

    nrm = jnp.sqrt(jnp.sum(X * X, axis=1, keepdims=True))
    Xn = X / jnp.maximum(nrm, 1e-8)
    S = jax.lax.dot_general(Xn, Xn, (((1,), (1,)), ((), ())),
                            preferred_element_type=jnp.float32)
    S = jnp.clip(S, -1.0 + 1e-8, 1.0 - 1e-8)

    rows = jax.lax.broadcasted_iota(jnp.int32, (B, B), 0)
    cols = jax.lax.broadcasted_iota(jnp.int32, (B, B), 1)
    eye = rows == cols
    same = labc == labr
    allowed = same & (~eye)
    masked = jnp.where(allowed, S, NEG)

    tc = jnp.max(masked, axis=1, keepdims=True)
    for _ in range(TOPK - 1):
        tc = jnp.max(jnp.where(masked < tc, masked, NEGINF),
                     axis=1, keepdims=True)
    eyef = jnp.where(eye, 1.0, 0.0)
    tr = jnp.transpose(tc)

    msym = ((S >= tc) | (S >= tr)) & allowed
    Mf = jnp.where(msym, 1.0, 0.0).astype(jnp.float32)

    A = Mf * jnp.maximum(S, 0.0) + 1e-6 * eyef
    dinv_c = jax.lax.rsqrt(jnp.maximum(jnp.sum(A, axis=1, keepdims=True), 1e-8))
    dinv_r = jax.lax.rsqrt(jnp.maximum(jnp.sum(A, axis=0, keepdims=True), 1e-8))
    A_norm = A * dinv_c * dinv_r

    W1t = w1t_ref[0]
    W2t = w2t_ref[0]
    gam = g_ref[0]
    bet = b_ref[0]
    bf = jnp.bfloat16
    Ab = A_norm.astype(bf)
    Z = jnp.dot(Ab, jnp.dot(X.astype(bf), W1t,
                            preferred_element_type=jnp.float32).astype(bf),
                preferred_element_type=jnp.float32)
    mu = jnp.mean(Z, axis=0, keepdims=True)
    var = jnp.mean((Z - mu) ** 2, axis=0, keepdims=True)
    Z = (Z - mu) * jax.lax.rsqrt(var + 1e-5) * gam + bet
    Z = jnp.maximum(Z, 0.0)
    Z = jnp.dot(Ab, jnp.dot(Z.astype(bf), W2t,
                            preferred_element_type=jnp.float32).astype(bf),
                preferred_element_type=jnp.float32) + X

    Pj = jnp.dot(Z.astype(bf), wpt_ref[...],
                 preferred_element_type=jnp.float32)
    pn = jnp.sqrt(jnp.sum(Pj * Pj, axis=1, keepdims=True))
    Pn = Pj / jnp.maximum(pn, 1e-8)

    @pl.when(i == 0)
    def _init():
        out_ref[...] = jnp.zeros_like(out_ref)

    @pl.when(i > 0)
    def _acc():
        Me = jnp.maximum(mprev[...], Mf)
        num = jnp.sum((kprev[...] - A_norm) ** 2 * Me)
        den = jnp.maximum(jnp.sum(Me), 1e-8)
        lk = num / den
        lz = jnp.sum((pprev[...] - Pn) ** 2) / (Pn.shape[0] * Pn.shape[1])
        lane = jax.lax.broadcasted_iota(jnp.int32, out_ref.shape, 1)
        out_ref[...] += (jnp.where(lane == 0, lk, 0.0)
                         + jnp.where(lane == 1, lz, 0.0)
                         + jnp.where(lane == 2, 64.0 * lk + 16.0 * lz, 0.0))

    kprev[...] = A_norm
    mprev[...] = Mf
    pprev[...] = Pn


def kernel(feats_final, labels, W1s, W2s, gammas, betas, Wproj):
    L, B, D = feats_final.shape
    P = Wproj.shape[0]

    labc = labels.astype(jnp.int32).reshape(B, 1)
    labr = labels.astype(jnp.int32).reshape(1, B)
    W1t = W1s.transpose(0, 2, 1).astype(jnp.bfloat16)
    W2t = W2s.transpose(0, 2, 1).astype(jnp.bfloat16)
    Wpt = Wproj.T.astype(jnp.bfloat16)
    g3 = gammas.reshape(L, 1, D)
    b3 = betas.reshape(L, 1, D)

    acc = pl.pallas_call(
        _body,
        grid=(L,),
        in_specs=[
            pl.BlockSpec((1, B, D), lambda i: (i, 0, 0)),
            pl.BlockSpec((B, 1), lambda i: (0, 0)),
            pl.BlockSpec((1, B), lambda i: (0, 0)),
            pl.BlockSpec((1, D, D), lambda i: (i, 0, 0)),
            pl.BlockSpec((1, D, D), lambda i: (i, 0, 0)),
            pl.BlockSpec((1, 1, D), lambda i: (i, 0, 0)),
            pl.BlockSpec((1, 1, D), lambda i: (i, 0, 0)),
            pl.BlockSpec((D, P), lambda i: (0, 0)),
        ],
        out_specs=pl.BlockSpec((1, 128), lambda i: (0, 0)),
        out_shape=jax.ShapeDtypeStruct((1, 128), jnp.float32),
        scratch_shapes=[
            pltpu.VMEM((B, B), jnp.float32),
            pltpu.VMEM((B, B), jnp.float32),
            pltpu.VMEM((B, P), jnp.float32),
        ],
    )(feats_final, labc, labr, W1t, W2t, g3, b3, Wpt)

    return (acc[0, 0], acc[0, 1], acc[0, 2])


# speedup vs baseline: 1.2384x; 1.1467x over previous
import jax
import jax.numpy as jnp
from jax.experimental import pallas as pl
from jax.experimental.pallas import tpu as pltpu

TOPK = 8
NEG = -1e9
NEGINF = -3.0e38


def _body(x_ref, labc_ref, labr_ref, w1t_ref, w2t_ref, g_ref, b_ref,
          wpt_ref, out_ref, kprev, mprev, pprev):
    i = pl.program_id(0)
    X = x_ref[0]
    B = X.shape[0]
    labc = labc_ref[...]
    labr = labr_ref[...]

    nrm = jnp.sqrt(jnp.sum(X * X, axis=1, keepdims=True))
    Xn = X / jnp.maximum(nrm, 1e-8)
    S = jax.lax.dot_general(Xn, Xn, (((1,), (1,)), ((), ())),
                            preferred_element_type=jnp.float32)
    S = jnp.clip(S, -1.0 + 1e-8, 1.0 - 1e-8)

    rows = jax.lax.broadcasted_iota(jnp.int32, (B, B), 0)
    cols = jax.lax.broadcasted_iota(jnp.int32, (B, B), 1)
    eye = rows == cols
    same = labc == labr
    allowed = same & (~eye)
    masked = jnp.where(allowed, S, NEG)

    tc = jnp.max(masked, axis=1, keepdims=True)
    for _ in range(TOPK - 1):
        tc = jnp.max(jnp.where(masked < tc, masked, NEGINF),
                     axis=1, keepdims=True)
    eyef = jnp.where(eye, 1.0, 0.0)
    tr = jnp.transpose(tc)

    msym = ((S >= tc) | (S >= tr)) & allowed
    Mf = jnp.where(msym, 1.0, 0.0).astype(jnp.float32)

    A = Mf * jnp.maximum(S, 0.0) + 1e-6 * eyef
    dinv_c = jax.lax.rsqrt(jnp.maximum(jnp.sum(A, axis=1, keepdims=True), 1e-8))
    dinv_r = jax.lax.rsqrt(jnp.maximum(jnp.sum(A, axis=0, keepdims=True), 1e-8))
    A_norm = A * dinv_c * dinv_r

    cdims = (((1,), (1,)), ((), ()))
    W1b = w1t_ref[0].astype(jnp.bfloat16)
    W2b = w2t_ref[0].astype(jnp.bfloat16)
    gam = g_ref[0]
    bet = b_ref[0]
    bf = jnp.bfloat16
    Ab = A_norm.astype(bf)
    Z = jnp.dot(Ab, jax.lax.dot_general(X.astype(bf), W1b, cdims,
                            preferred_element_type=jnp.float32).astype(bf),
                preferred_element_type=jnp.float32)
    mu = jnp.mean(Z, axis=0, keepdims=True)
    var = jnp.mean((Z - mu) ** 2, axis=0, keepdims=True)
    Z = (Z - mu) * jax.lax.rsqrt(var + 1e-5) * gam + bet
    Z = jnp.maximum(Z, 0.0)
    Z = jnp.dot(Ab, jax.lax.dot_general(Z.astype(bf), W2b, cdims,
                            preferred_element_type=jnp.float32).astype(bf),
                preferred_element_type=jnp.float32) + X

    Pj = jax.lax.dot_general(Z.astype(bf), wpt_ref[...].astype(jnp.bfloat16),
                 cdims, preferred_element_type=jnp.float32)
    pn = jnp.sqrt(jnp.sum(Pj * Pj, axis=1, keepdims=True))
    Pn = Pj / jnp.maximum(pn, 1e-8)

    @pl.when(i == 0)
    def _init():
        out_ref[...] = jnp.zeros_like(out_ref)

    @pl.when(i > 0)
    def _acc():
        Me = jnp.maximum(mprev[...], Mf)
        num = jnp.sum((kprev[...] - A_norm) ** 2 * Me)
        den = jnp.maximum(jnp.sum(Me), 1e-8)
        lk = num / den
        lz = jnp.sum((pprev[...] - Pn) ** 2) / (Pn.shape[0] * Pn.shape[1])
        lane = jax.lax.broadcasted_iota(jnp.int32, out_ref.shape, 1)
        out_ref[...] += (jnp.where(lane == 0, lk, 0.0)
                         + jnp.where(lane == 1, lz, 0.0)
                         + jnp.where(lane == 2, 64.0 * lk + 16.0 * lz, 0.0))

    kprev[...] = A_norm
    mprev[...] = Mf
    pprev[...] = Pn


def kernel(feats_final, labels, W1s, W2s, gammas, betas, Wproj):
    L, B, D = feats_final.shape
    P = Wproj.shape[0]

    labc = labels.astype(jnp.int32).reshape(B, 1)
    labr = labels.astype(jnp.int32).reshape(1, B)
    W1t = W1s
    W2t = W2s
    Wpt = Wproj
    g3 = gammas.reshape(L, 1, D)
    b3 = betas.reshape(L, 1, D)

    acc = pl.pallas_call(
        _body,
        grid=(L,),
        in_specs=[
            pl.BlockSpec((1, B, D), lambda i: (i, 0, 0)),
            pl.BlockSpec((B, 1), lambda i: (0, 0)),
            pl.BlockSpec((1, B), lambda i: (0, 0)),
            pl.BlockSpec((1, D, D), lambda i: (i, 0, 0)),
            pl.BlockSpec((1, D, D), lambda i: (i, 0, 0)),
            pl.BlockSpec((1, 1, D), lambda i: (i, 0, 0)),
            pl.BlockSpec((1, 1, D), lambda i: (i, 0, 0)),
            pl.BlockSpec((P, D), lambda i: (0, 0)),
        ],
        out_specs=pl.BlockSpec((1, 128), lambda i: (0, 0)),
        out_shape=jax.ShapeDtypeStruct((1, 128), jnp.float32),
        scratch_shapes=[
            pltpu.VMEM((B, B), jnp.float32),
            pltpu.VMEM((B, B), jnp.float32),
            pltpu.VMEM((B, P), jnp.float32),
        ],
    )(feats_final, labc, labr, W1t, W2t, g3, b3, Wpt)

    return (acc[0, 0], acc[0, 1], acc[0, 2])


# speedup vs baseline: 1.2420x; 1.0029x over previous
import jax
import jax.numpy as jnp
from jax.experimental import pallas as pl
from jax.experimental.pallas import tpu as pltpu

TOPK = 8
NEG = -1e9
NEGINF = -3.0e38


def _body(x_ref, labc_ref, labr_ref, w1t_ref, w2t_ref, g_ref, b_ref,
          wpt_ref, out_ref, kprev, mprev, pprev):
    i = pl.program_id(0)
    X = x_ref[0]
    B = X.shape[0]
    labc = labc_ref[...]
    labr = labr_ref[...]

    nrm = jnp.sqrt(jnp.sum(X * X, axis=1, keepdims=True))
    Xn = X / jnp.maximum(nrm, 1e-8)
    S = jax.lax.dot_general(Xn, Xn, (((1,), (1,)), ((), ())),
                            preferred_element_type=jnp.float32)
    S = jnp.clip(S, -1.0 + 1e-8, 1.0 - 1e-8)

    rows = jax.lax.broadcasted_iota(jnp.int32, (B, B), 0)
    cols = jax.lax.broadcasted_iota(jnp.int32, (B, B), 1)
    eye = rows == cols
    same = labc == labr
    allowed = same & (~eye)
    masked = jnp.where(allowed, S, NEG)

    tc = jnp.max(masked, axis=1, keepdims=True)
    for _ in range(TOPK - 1):
        tc = jnp.max(jnp.where(masked < tc, masked, NEGINF),
                     axis=1, keepdims=True)
    eyef = jnp.where(eye, 1.0, 0.0)
    tr = jnp.transpose(tc)

    msym = ((S >= tc) | (S >= tr)) & allowed
    Mf = jnp.where(msym, 1.0, 0.0).astype(jnp.float32)

    A = Mf * jnp.maximum(S, 0.0) + 1e-6 * eyef
    dinv_c = jax.lax.rsqrt(jnp.maximum(jnp.sum(A, axis=1, keepdims=True), 1e-8))
    dinv_r = jax.lax.rsqrt(jnp.maximum(jnp.sum(A, axis=0, keepdims=True), 1e-8))
    A_norm = A * dinv_c * dinv_r

    cdims = (((1,), (1,)), ((), ()))
    W1b = w1t_ref[0].astype(jnp.bfloat16)
    W2b = w2t_ref[0].astype(jnp.bfloat16)
    gam = g_ref[0]
    bet = b_ref[0]
    bf = jnp.bfloat16
    Ab = A_norm.astype(bf)
    Z = jnp.dot(Ab, jax.lax.dot_general(X.astype(bf), W1b, cdims,
                            preferred_element_type=jnp.float32).astype(bf),
                preferred_element_type=jnp.float32)
    mu = jnp.mean(Z, axis=0, keepdims=True)
    var = jnp.mean((Z - mu) ** 2, axis=0, keepdims=True)
    Z = (Z - mu) * jax.lax.rsqrt(var + 1e-5) * gam + bet
    Z = jnp.maximum(Z, 0.0)
    Z = jnp.dot(Ab, jax.lax.dot_general(Z.astype(bf), W2b, cdims,
                            preferred_element_type=jnp.float32).astype(bf),
                preferred_element_type=jnp.float32) + X

    Pj = jax.lax.dot_general(Z.astype(bf), wpt_ref[...].astype(jnp.bfloat16),
                 cdims, preferred_element_type=jnp.float32)
    pn = jnp.sqrt(jnp.sum(Pj * Pj, axis=1, keepdims=True))
    Pn = Pj / jnp.maximum(pn, 1e-8)

    @pl.when(i == 0)
    def _init():
        out_ref[...] = jnp.zeros_like(out_ref)

    @pl.when(i > 0)
    def _acc():
        Me = jnp.maximum(mprev[...], Mf)
        num = jnp.sum((kprev[...] - A_norm) ** 2 * Me)
        den = jnp.maximum(jnp.sum(Me), 1e-8)
        lk = num / den
        lz = jnp.sum((pprev[...] - Pn) ** 2) / (Pn.shape[0] * Pn.shape[1])
        lane = jax.lax.broadcasted_iota(jnp.int32, out_ref.shape, 1)
        out_ref[...] += (jnp.where(lane == 0, lk, 0.0)
                         + jnp.where(lane == 1, lz, 0.0)
                         + jnp.where(lane == 2, 64.0 * lk + 16.0 * lz, 0.0))

    @pl.when(i < pl.num_programs(0) - 1)
    def _save():
        kprev[...] = A_norm
        mprev[...] = Mf
        pprev[...] = Pn


def kernel(feats_final, labels, W1s, W2s, gammas, betas, Wproj):
    L, B, D = feats_final.shape
    P = Wproj.shape[0]

    labc = labels.astype(jnp.int32).reshape(B, 1)
    labr = labels.astype(jnp.int32).reshape(1, B)
    W1t = W1s
    W2t = W2s
    Wpt = Wproj
    g3 = gammas.reshape(L, 1, D)
    b3 = betas.reshape(L, 1, D)

    acc = pl.pallas_call(
        _body,
        grid=(L,),
        in_specs=[
            pl.BlockSpec((1, B, D), lambda i: (i, 0, 0)),
            pl.BlockSpec((B, 1), lambda i: (0, 0)),
            pl.BlockSpec((1, B), lambda i: (0, 0)),
            pl.BlockSpec((1, D, D), lambda i: (i, 0, 0)),
            pl.BlockSpec((1, D, D), lambda i: (i, 0, 0)),
            pl.BlockSpec((1, 1, D), lambda i: (i, 0, 0)),
            pl.BlockSpec((1, 1, D), lambda i: (i, 0, 0)),
            pl.BlockSpec((P, D), lambda i: (0, 0)),
        ],
        out_specs=pl.BlockSpec((1, 128), lambda i: (0, 0)),
        out_shape=jax.ShapeDtypeStruct((1, 128), jnp.float32),
        scratch_shapes=[
            pltpu.VMEM((B, B), jnp.float32),
            pltpu.VMEM((B, B), jnp.float32),
            pltpu.VMEM((B, P), jnp.float32),
        ],
    )(feats_final, labc, labr, W1t, W2t, g3, b3, Wpt)

    return (acc[0, 0], acc[0, 1], acc[0, 2])
